# baseline (speedup 1.0000x reference)
import jax
import jax.numpy as jnp
from jax.experimental import pallas as pl
from jax.experimental.pallas import tpu as pltpu

_N = 197
_C = 384
_H = 6
_DH = 64
_GS = 49
_G = 4
_TOPK = 2


def _attn_kernel(x_ref, wqkv_ref, bqkv_ref, wproj_ref, bproj_ref, o_ref):
    xb = x_ref[0]
    qkv = (
        jax.lax.dot_general(
            xb, wqkv_ref[...], (((1,), (1,)), ((), ())),
            preferred_element_type=jnp.float32,
        )
        + bqkv_ref[...]
    )
    scale = _DH ** -0.5

    row1 = jax.lax.broadcasted_iota(jnp.int32, (_N, 1), 0)
    grprow = (row1 - 1) // _GS
    rowmasks = [(grprow == g).astype(jnp.float32) for g in range(_G)]
    ones_col = jnp.ones((_N, 1), jnp.float32)
    rank0 = jnp.where(row1 == 0, -float(_G), 0.0)

    k_all = qkv[:, _C:2 * _C]
    gstats = []
    for g in range(_G):
        kg = k_all[1 + g * _GS:1 + (g + 1) * _GS, :]
        gstats.append((jnp.max(kg, axis=0, keepdims=True),
                       jnp.min(kg, axis=0, keepdims=True)))

    outs = []
    for h in range(_H):
        lo = h * _DH
        hi = lo + _DH
        q = qkv[:, lo:hi]
        k = qkv[:, _C + lo:_C + hi]
        v = qkv[:, 2 * _C + lo:2 * _C + hi]

        qpos = q >= 0.0
        scols = []
        for g in range(_G):
            gmax, gmin = gstats[g]
            ms = jnp.where(qpos, gmax[:, lo:hi], gmin[:, lo:hi])
            scols.append(jnp.sum(q * ms, axis=1, keepdims=True))

        a = {}
        for j in range(_G):
            for g in range(j + 1, _G):
                a[(j, g)] = (scols[j] >= scols[g]).astype(jnp.float32)
        sel = []
        for g in range(_G):
            rank = rank0 + float(_G - 1 - g)
            for j in range(g):
                rank = rank + a[(j, g)]
            for j in range(g + 1, _G):
                rank = rank - a[(g, j)]
            sel.append((rank < _TOPK).astype(jnp.float32))

        logits = jax.lax.dot_general(
            q * scale, k, (((1,), (1,)), ((), ())),
            preferred_element_type=jnp.float32,
        )
        mrow = jnp.maximum(jnp.max(logits, axis=1, keepdims=True), 0.0)
        e0 = jnp.exp(logits - mrow)
        r = jnp.exp(-mrow)

        vx = jnp.concatenate([v, ones_col], axis=1)
        acc = e0[:, 0:1] * vx[0:1, :]
        for g in range(_G):
            vg = vx * rowmasks[g]
            ag = jnp.dot(e0, vg, preferred_element_type=jnp.float32)
            vsum = jnp.sum(vg, axis=0, keepdims=True)
            ug = (1.0 - sel[g]) * r
            acc = acc + sel[g] * ag + ug * vsum
        outs.append(acc[:, 0:_DH] * (1.0 / acc[:, _DH:_DH + 1]))

    out = jnp.concatenate(outs, axis=1)
    o_ref[0] = (
        jax.lax.dot_general(
            out, wproj_ref[...], (((1,), (1,)), ((), ())),
            preferred_element_type=jnp.float32,
        )
        + bproj_ref[...]
    )


def kernel(x, Wqkv, bqkv, Wproj, bproj):
    Bsz = x.shape[0]
    bqkv2 = bqkv.reshape(1, -1)
    bproj2 = bproj.reshape(1, -1)
    return pl.pallas_call(
        _attn_kernel,
        grid=(Bsz,),
        in_specs=[
            pl.BlockSpec((1, _N, _C), lambda b: (b, 0, 0)),
            pl.BlockSpec((3 * _C, _C), lambda b: (0, 0)),
            pl.BlockSpec((1, 3 * _C), lambda b: (0, 0)),
            pl.BlockSpec((_C, _C), lambda b: (0, 0)),
            pl.BlockSpec((1, _C), lambda b: (0, 0)),
        ],
        out_specs=pl.BlockSpec((1, _N, _C), lambda b: (b, 0, 0)),
        out_shape=jax.ShapeDtypeStruct(x.shape, x.dtype),
        compiler_params=pltpu.CompilerParams(
            dimension_semantics=("parallel",),
        ),
    )(x, Wqkv, bqkv2, Wproj, bproj2)


# speedup vs baseline: 1.0569x; 1.0569x over previous
import jax
import jax.numpy as jnp
from jax.experimental import pallas as pl
from jax.experimental.pallas import tpu as pltpu

_N = 197
_C = 384
_H = 6
_DH = 64
_GS = 49
_G = 4
_TOPK = 2


def _attn_kernel(x_ref, wqkv_ref, bqkv_ref, wproj_ref, bproj_ref, o_ref):
    xb = x_ref[0]
    qkv = (
        jax.lax.dot_general(
            xb, wqkv_ref[...], (((1,), (1,)), ((), ())),
            preferred_element_type=jnp.float32,
        )
        + bqkv_ref[...]
    )
    scale = _DH ** -0.5

    row1 = jax.lax.broadcasted_iota(jnp.int32, (_N, 1), 0)
    grprow = (row1 - 1) // _GS
    rowmasks = [(grprow == g).astype(jnp.float32) for g in range(_G)]
    ones_col = jnp.ones((_N, 1), jnp.float32)
    rank0 = jnp.where(row1 == 0, -float(_G), 0.0)

    k_all = qkv[:, _C:2 * _C]
    gstats = []
    for g in range(_G):
        kg = k_all[1 + g * _GS:1 + (g + 1) * _GS, :]
        gstats.append((jnp.max(kg, axis=0, keepdims=True),
                       jnp.min(kg, axis=0, keepdims=True)))

    outs = []
    for h in range(_H):
        lo = h * _DH
        hi = lo + _DH
        q = qkv[:, lo:hi]
        k = qkv[:, _C + lo:_C + hi]
        v = qkv[:, 2 * _C + lo:2 * _C + hi]

        qpos = q >= 0.0
        scols = []
        for g in range(_G):
            gmax, gmin = gstats[g]
            ms = jnp.where(qpos, gmax[:, lo:hi], gmin[:, lo:hi])
            scols.append(jnp.sum(q * ms, axis=1, keepdims=True))

        a = {}
        for j in range(_G):
            for g in range(j + 1, _G):
                a[(j, g)] = (scols[j] >= scols[g]).astype(jnp.float32)
        sel = []
        for g in range(_G):
            rank = rank0 + float(_G - 1 - g)
            for j in range(g):
                rank = rank + a[(j, g)]
            for j in range(g + 1, _G):
                rank = rank - a[(g, j)]
            sel.append((rank < _TOPK).astype(jnp.float32))

        logits = jax.lax.dot_general(
            q * scale, k, (((1,), (1,)), ((), ())),
            preferred_element_type=jnp.float32,
        )
        mrow = jnp.maximum(jnp.max(logits, axis=1, keepdims=True), 0.0)
        e0 = jnp.exp(logits - mrow)
        r = jnp.exp(-mrow)

        vx = jnp.concatenate([v, ones_col], axis=1)
        e0b = e0.astype(jnp.bfloat16)
        acc = e0[:, 0:1] * vx[0:1, :]
        for g in range(_G):
            vg = vx * rowmasks[g]
            ag = jnp.dot(e0b, vg.astype(jnp.bfloat16),
                         preferred_element_type=jnp.float32)
            vsum = jnp.sum(vg, axis=0, keepdims=True)
            ug = (1.0 - sel[g]) * r
            acc = acc + sel[g] * ag + ug * vsum
        outs.append(acc[:, 0:_DH] * (1.0 / acc[:, _DH:_DH + 1]))

    out = jnp.concatenate(outs, axis=1)
    o_ref[0] = (
        jax.lax.dot_general(
            out, wproj_ref[...], (((1,), (1,)), ((), ())),
            preferred_element_type=jnp.float32,
        )
        + bproj_ref[...]
    )


def kernel(x, Wqkv, bqkv, Wproj, bproj):
    Bsz = x.shape[0]
    bqkv2 = bqkv.reshape(1, -1)
    bproj2 = bproj.reshape(1, -1)
    return pl.pallas_call(
        _attn_kernel,
        grid=(Bsz,),
        in_specs=[
            pl.BlockSpec((1, _N, _C), lambda b: (b, 0, 0)),
            pl.BlockSpec((3 * _C, _C), lambda b: (0, 0)),
            pl.BlockSpec((1, 3 * _C), lambda b: (0, 0)),
            pl.BlockSpec((_C, _C), lambda b: (0, 0)),
            pl.BlockSpec((1, _C), lambda b: (0, 0)),
        ],
        out_specs=pl.BlockSpec((1, _N, _C), lambda b: (b, 0, 0)),
        out_shape=jax.ShapeDtypeStruct(x.shape, x.dtype),
        compiler_params=pltpu.CompilerParams(
            dimension_semantics=("parallel",),
        ),
    )(x, Wqkv, bqkv2, Wproj, bproj2)
